# baseline (speedup 1.0000x reference)
import functools

import jax
import jax.numpy as jnp
from jax import lax
from jax.experimental import pallas as pl
from jax.experimental.pallas import tpu as pltpu
from jax.experimental.pallas import tpu_sc as plsc

B, M, S, E = 1024, 50, 20, 32

NC = 2
NS = 16
NW = NC * NS

ROWS = B * M
RPW = ROWS // NW
CHUNK = 32
NCH = RPW // CHUNK
IPC = CHUNK * S
NSTR = IPC // 128


def _seg_gather_body(ctx_hbm, t0_hbm, t1_hbm, t2_hbm, o0_hbm, o1_hbm, o2_hbm,
                     idx_v, rows_v, out_v, gsem0, gsem1):
  wid = lax.axis_index("s") * NC + lax.axis_index("c")
  idx_base = wid * (NCH * IPC)

  def load_idx(c, buf):
    pltpu.sync_copy(ctx_hbm.at[pl.ds(idx_base + c * IPC, IPC)],
                    idx_v.at[buf])

  def fire(tbl, buf, sem):
    for k in range(NSTR):
      pltpu.async_copy(tbl.at[idx_v.at[buf, pl.ds(k * 128, 128)]],
                       rows_v.at[buf, pl.ds(k * 128, 128), :],
                       sem)

  def drain(tbl, buf, sem):
    for k in range(NSTR):
      pltpu.make_async_copy(tbl.at[pl.ds(0, 128), :],
                            rows_v.at[buf, pl.ds(k * 128, 128), :],
                            sem).wait()

  hi_mask = jnp.full((16,), -65536, dtype=jnp.int32)

  def widen(xi):
    lo = lax.bitcast_convert_type(xi << 16, jnp.float32)
    hi = lax.bitcast_convert_type(xi & hi_mask, jnp.float32)
    return lo, hi

  def compute_store(o_hbm, c, buf):
    def row(r, carry):
      base = r * S
      acc0, acc1 = widen(rows_v[buf, base, :])
      for s in range(1, S):
        lo, hi = widen(rows_v[buf, base + s, :])
        acc0 = acc0 + lo
        acc1 = acc1 + hi
      out_v[r, pl.ds(0, 16)] = acc0
      out_v[r, pl.ds(16, 16)] = acc1
      return carry
    lax.fori_loop(0, CHUNK, row, 0)
    pltpu.sync_copy(out_v, o_hbm.at[pl.ds(wid * RPW + c * CHUNK, CHUNK), :])

  for tbl, o_hbm in ((t0_hbm, o0_hbm), (t1_hbm, o1_hbm), (t2_hbm, o2_hbm)):
    load_idx(0, 0)
    fire(tbl, 0, gsem0)

    def pair(j, carry, tbl=tbl, o_hbm=o_hbm):
      c0 = 2 * j
      load_idx(c0 + 1, 1)
      fire(tbl, 1, gsem1)
      drain(tbl, 0, gsem0)
      compute_store(o_hbm, c0, 0)
      load_idx(c0 + 2, 0)
      fire(tbl, 0, gsem0)
      drain(tbl, 1, gsem1)
      compute_store(o_hbm, c0 + 1, 1)
      return carry

    lax.fori_loop(0, NCH // 2 - 1, pair, 0)
    c0 = NCH - 2
    load_idx(c0 + 1, 1)
    fire(tbl, 1, gsem1)
    drain(tbl, 0, gsem0)
    compute_store(o_hbm, c0, 0)
    drain(tbl, 1, gsem1)
    compute_store(o_hbm, c0 + 1, 1)


@functools.cache
def _seg_gather():
  mesh = plsc.VectorSubcoreMesh(
      core_axis_name="c", subcore_axis_name="s",
      num_cores=NC, num_subcores=NS)
  return pl.kernel(
      _seg_gather_body,
      mesh=mesh,
      out_type=[jax.ShapeDtypeStruct((ROWS, E), jnp.float32)] * 3,
      scratch_types=[
          pltpu.VMEM((2, IPC), jnp.int32),
          pltpu.VMEM((2, IPC, E // 2), jnp.int32),
          pltpu.VMEM((CHUNK, E), jnp.float32),
          pltpu.SemaphoreType.DMA,
          pltpu.SemaphoreType.DMA,
      ],
      compiler_params=pltpu.CompilerParams(use_tc_tiling_on_sc=False),
  )


def _hops_body(g1, g2, g3, o):
  g1v = g1[...]
  q1 = jnp.sum(g1v, axis=1) * (1.0 / M)
  p1 = jnp.sum(g1v * q1[:, None, :], axis=2)
  a1 = jax.nn.softmax(p1, axis=1)
  g2v = g2[...]
  q2 = q1 + jnp.sum(a1[:, :, None] * g2v, axis=1)
  p2 = jnp.sum(g2v * q2[:, None, :], axis=2)
  a2 = jax.nn.softmax(p2, axis=1)
  o[...] = q2 + jnp.sum(a2[:, :, None] * g3[...], axis=1)


def _hops(G1, G2, G3):
  BB = 256
  spec3 = pl.BlockSpec((BB, M, E), lambda i: (i, 0, 0))
  return pl.pallas_call(
      _hops_body,
      grid=(B // BB,),
      in_specs=[spec3, spec3, spec3],
      out_specs=pl.BlockSpec((BB, E), lambda i: (i, 0)),
      out_shape=jax.ShapeDtypeStruct((B, E), jnp.float32),
  )(G1, G2, G3)


def kernel(context, A0, A1, A2, C_last):
  del A0
  ctx = context.reshape(-1)

  def pack_table(t):
    tb = t.astype(jnp.bfloat16).reshape(t.shape[0], E // 2, 2)
    return lax.bitcast_convert_type(tb, jnp.int32)

  G1, G2, G3 = _seg_gather()(
      ctx, pack_table(A1), pack_table(A2), pack_table(C_last))
  q = _hops(G1.reshape(B, M, E), G2.reshape(B, M, E), G3.reshape(B, M, E))
  inv_perm = jnp.argsort(
      jnp.concatenate([jnp.arange(0, E, 2), jnp.arange(1, E, 2)]))
  return q[:, inv_perm]


# speedup vs baseline: 1.2160x; 1.2160x over previous
import functools

import jax
import jax.numpy as jnp
from jax import lax
from jax.experimental import pallas as pl
from jax.experimental.pallas import tpu as pltpu
from jax.experimental.pallas import tpu_sc as plsc

B, M, S, E = 1024, 50, 20, 32
NT = 3
PW = NT * E // 2

NC = 2
NS = 16
NW = NC * NS

ROWS = B * M
RPW = ROWS // NW
CHUNK = 32
NCH = RPW // CHUNK
IPC = CHUNK * S
NSTR = IPC // 128


def _seg_gather_body(ctx_hbm, tbl_hbm, o0_hbm, o1_hbm, o2_hbm,
                     idx_v, rows_v, out_v, gsem0, gsem1):
  wid = lax.axis_index("s") * NC + lax.axis_index("c")
  idx_base = wid * (NCH * IPC)
  o_hbms = (o0_hbm, o1_hbm, o2_hbm)

  def load_idx(c, buf):
    pltpu.sync_copy(ctx_hbm.at[pl.ds(idx_base + c * IPC, IPC)],
                    idx_v.at[buf])

  def fire(buf, sem):
    for k in range(NSTR):
      pltpu.async_copy(tbl_hbm.at[idx_v.at[buf, pl.ds(k * 128, 128)]],
                       rows_v.at[buf, pl.ds(k * 128, 128), :],
                       sem)

  def drain(buf, sem):
    for k in range(NSTR):
      pltpu.make_async_copy(tbl_hbm.at[pl.ds(0, 128), :],
                            rows_v.at[buf, pl.ds(k * 128, 128), :],
                            sem).wait()

  hi_mask = jnp.full((16,), -65536, dtype=jnp.int32)

  def widen(xi):
    lo = lax.bitcast_convert_type(xi << 16, jnp.float32)
    hi = lax.bitcast_convert_type(xi & hi_mask, jnp.float32)
    return lo, hi

  def compute_store(c, buf):
    def row(r, carry):
      base = r * S
      for t in range(NT):
        acc0, acc1 = widen(rows_v[buf, base, pl.ds(16 * t, 16)])
        for s in range(1, S):
          lo, hi = widen(rows_v[buf, base + s, pl.ds(16 * t, 16)])
          acc0 = acc0 + lo
          acc1 = acc1 + hi
        out_v[t, r, pl.ds(0, 16)] = acc0
        out_v[t, r, pl.ds(16, 16)] = acc1
      return carry
    lax.fori_loop(0, CHUNK, row, 0)
    for t in range(NT):
      pltpu.sync_copy(out_v.at[t],
                      o_hbms[t].at[pl.ds(wid * RPW + c * CHUNK, CHUNK), :])

  load_idx(0, 0)
  fire(0, gsem0)

  def pair(j, carry):
    c0 = 2 * j
    load_idx(c0 + 1, 1)
    fire(1, gsem1)
    drain(0, gsem0)
    compute_store(c0, 0)
    load_idx(c0 + 2, 0)
    fire(0, gsem0)
    drain(1, gsem1)
    compute_store(c0 + 1, 1)
    return carry

  lax.fori_loop(0, NCH // 2 - 1, pair, 0)
  c0 = NCH - 2
  load_idx(c0 + 1, 1)
  fire(1, gsem1)
  drain(0, gsem0)
  compute_store(c0, 0)
  drain(1, gsem1)
  compute_store(c0 + 1, 1)


@functools.cache
def _seg_gather():
  mesh = plsc.VectorSubcoreMesh(
      core_axis_name="c", subcore_axis_name="s",
      num_cores=NC, num_subcores=NS)
  return pl.kernel(
      _seg_gather_body,
      mesh=mesh,
      out_type=[jax.ShapeDtypeStruct((ROWS, E), jnp.float32)] * NT,
      scratch_types=[
          pltpu.VMEM((2, IPC), jnp.int32),
          pltpu.VMEM((2, IPC, PW), jnp.int32),
          pltpu.VMEM((NT, CHUNK, E), jnp.float32),
          pltpu.SemaphoreType.DMA,
          pltpu.SemaphoreType.DMA,
      ],
      compiler_params=pltpu.CompilerParams(use_tc_tiling_on_sc=False),
  )


def _hops_body(g1, g2, g3, o):
  g1v = g1[...]
  q1 = jnp.sum(g1v, axis=1) * (1.0 / M)
  p1 = jnp.sum(g1v * q1[:, None, :], axis=2)
  a1 = jax.nn.softmax(p1, axis=1)
  g2v = g2[...]
  q2 = q1 + jnp.sum(a1[:, :, None] * g2v, axis=1)
  p2 = jnp.sum(g2v * q2[:, None, :], axis=2)
  a2 = jax.nn.softmax(p2, axis=1)
  o[...] = q2 + jnp.sum(a2[:, :, None] * g3[...], axis=1)


def _hops(G1, G2, G3):
  BB = 256
  spec3 = pl.BlockSpec((BB, M, E), lambda i: (i, 0, 0))
  return pl.pallas_call(
      _hops_body,
      grid=(B // BB,),
      in_specs=[spec3, spec3, spec3],
      out_specs=pl.BlockSpec((BB, E), lambda i: (i, 0)),
      out_shape=jax.ShapeDtypeStruct((B, E), jnp.float32),
  )(G1, G2, G3)


def kernel(context, A0, A1, A2, C_last):
  del A0
  ctx = context.reshape(-1)
  packed = lax.bitcast_convert_type(
      jnp.concatenate([A1, A2, C_last], axis=1)
      .astype(jnp.bfloat16)
      .reshape(A1.shape[0], NT * E // 2, 2),
      jnp.int32)
  G1, G2, G3 = _seg_gather()(ctx, packed)
  q = _hops(G1.reshape(B, M, E), G2.reshape(B, M, E), G3.reshape(B, M, E))
  inv_perm = jnp.argsort(
      jnp.concatenate([jnp.arange(0, E, 2), jnp.arange(1, E, 2)]))
  return q[:, inv_perm]


# speedup vs baseline: 1.5848x; 1.3033x over previous
import functools

import jax
import jax.numpy as jnp
from jax import lax
from jax.experimental import pallas as pl
from jax.experimental.pallas import tpu as pltpu
from jax.experimental.pallas import tpu_sc as plsc

B, M, S, E = 1024, 50, 20, 32
V = 100000
NT = 3
PW = NT * E // 2

NC = 2
NS = 16
NW = NC * NS

ROWS = B * M
RPW = ROWS // NW
CHUNK = 32
NCH = RPW // CHUNK
IPC = CHUNK * S
NSTR = IPC // 128


def _pack_body(a1, a2, a3, o):
  def rne16(x):
    xi = lax.bitcast_convert_type(x, jnp.int32)
    return lax.shift_right_logical(
        xi + 0x7FFF + ((lax.shift_right_logical(xi, 16)) & 1), 16)

  cols = []
  for ref in (a1, a2, a3):
    x = ref[...]
    lo = rne16(x[:, :16])
    hi = rne16(x[:, 16:])
    cols.append(lo | (hi << 16))
  o[...] = jnp.concatenate(cols, axis=1)


def _pack_tables(A1, A2, C_last):
  BR = 4000
  spec = pl.BlockSpec((BR, E), lambda i: (i, 0))
  return pl.pallas_call(
      _pack_body,
      grid=(V // BR,),
      in_specs=[spec, spec, spec],
      out_specs=pl.BlockSpec((BR, PW), lambda i: (i, 0)),
      out_shape=jax.ShapeDtypeStruct((V, PW), jnp.int32),
  )(A1, A2, C_last)


def _seg_gather_body(ctx_hbm, tbl_hbm, o_hbm,
                     idx_v, rows_v, out_v, gsem0, gsem1, isem):
  wid = lax.axis_index("s") * NC + lax.axis_index("c")
  idx_base = wid * (NCH * IPC)

  def load_idx(c, buf):
    pltpu.async_copy(ctx_hbm.at[pl.ds(idx_base + c * IPC, IPC)],
                     idx_v.at[buf], isem)

  def wait_idx(buf):
    pltpu.make_async_copy(ctx_hbm.at[pl.ds(0, IPC)], idx_v.at[buf],
                          isem).wait()

  def fire(buf, sem):
    for k in range(NSTR):
      pltpu.async_copy(tbl_hbm.at[idx_v.at[buf, pl.ds(k * 128, 128)]],
                       rows_v.at[buf, pl.ds(k * 128, 128), :],
                       sem)

  def drain(buf, sem):
    for k in range(NSTR):
      pltpu.make_async_copy(tbl_hbm.at[pl.ds(0, 128), :],
                            rows_v.at[buf, pl.ds(k * 128, 128), :],
                            sem).wait()

  hi_mask = jnp.full((16,), -65536, dtype=jnp.int32)

  def widen(xi):
    lo = lax.bitcast_convert_type(xi << 16, jnp.float32)
    hi = lax.bitcast_convert_type(xi & hi_mask, jnp.float32)
    return lo, hi

  def compute_store(c, buf):
    def row(r, carry):
      base = r * S
      for t in range(NT):
        acc0, acc1 = widen(rows_v[buf, base, pl.ds(16 * t, 16)])
        for s in range(1, S):
          lo, hi = widen(rows_v[buf, base + s, pl.ds(16 * t, 16)])
          acc0 = acc0 + lo
          acc1 = acc1 + hi
        out_v[t, r, pl.ds(0, 16)] = acc0
        out_v[t, r, pl.ds(16, 16)] = acc1
      return carry
    lax.fori_loop(0, CHUNK, row, 0)
    pltpu.sync_copy(out_v,
                    o_hbm.at[:, pl.ds(wid * RPW + c * CHUNK, CHUNK), :])

  pltpu.sync_copy(ctx_hbm.at[pl.ds(idx_base, IPC)], idx_v.at[0])
  fire(0, gsem0)
  load_idx(1, 1)

  def pair(j, carry):
    c0 = 2 * j
    wait_idx(1)
    fire(1, gsem1)
    drain(0, gsem0)
    load_idx(c0 + 2, 0)
    compute_store(c0, 0)
    wait_idx(0)
    fire(0, gsem0)
    drain(1, gsem1)
    load_idx(c0 + 3, 1)
    compute_store(c0 + 1, 1)
    return carry

  lax.fori_loop(0, NCH // 2 - 1, pair, 0)
  c0 = NCH - 2
  wait_idx(1)
  fire(1, gsem1)
  drain(0, gsem0)
  compute_store(c0, 0)
  drain(1, gsem1)
  compute_store(c0 + 1, 1)


@functools.cache
def _seg_gather():
  mesh = plsc.VectorSubcoreMesh(
      core_axis_name="c", subcore_axis_name="s",
      num_cores=NC, num_subcores=NS)
  return pl.kernel(
      _seg_gather_body,
      mesh=mesh,
      out_type=jax.ShapeDtypeStruct((NT, ROWS, E), jnp.float32),
      scratch_types=[
          pltpu.VMEM((2, IPC), jnp.int32),
          pltpu.VMEM((2, IPC, PW), jnp.int32),
          pltpu.VMEM((NT, CHUNK, E), jnp.float32),
          pltpu.SemaphoreType.DMA,
          pltpu.SemaphoreType.DMA,
          pltpu.SemaphoreType.DMA,
      ],
      compiler_params=pltpu.CompilerParams(use_tc_tiling_on_sc=False),
  )


def _hops_body(g, o):
  g1v = g[0]
  q1 = jnp.sum(g1v, axis=1) * (1.0 / M)
  p1 = jnp.sum(g1v * q1[:, None, :], axis=2)
  a1 = jax.nn.softmax(p1, axis=1)
  g2v = g[1]
  q2 = q1 + jnp.sum(a1[:, :, None] * g2v, axis=1)
  p2 = jnp.sum(g2v * q2[:, None, :], axis=2)
  a2 = jax.nn.softmax(p2, axis=1)
  o[...] = q2 + jnp.sum(a2[:, :, None] * g[2], axis=1)


def _hops(G):
  BB = 128
  return pl.pallas_call(
      _hops_body,
      grid=(B // BB,),
      in_specs=[pl.BlockSpec((NT, BB, M, E), lambda i: (0, i, 0, 0))],
      out_specs=pl.BlockSpec((BB, E), lambda i: (i, 0)),
      out_shape=jax.ShapeDtypeStruct((B, E), jnp.float32),
  )(G)


def kernel(context, A0, A1, A2, C_last):
  del A0
  ctx = context.reshape(-1)
  packed = _pack_tables(A1, A2, C_last)
  G = _seg_gather()(ctx, packed)
  return _hops(G.reshape(NT, B, M, E))


# speedup vs baseline: 1.8621x; 1.1750x over previous
import functools

import jax
import jax.numpy as jnp
from jax import lax
from jax.experimental import pallas as pl
from jax.experimental.pallas import tpu as pltpu
from jax.experimental.pallas import tpu_sc as plsc

B, M, S, E = 1024, 50, 20, 32
V = 100000
NT = 3
PW = NT * E // 2

NC = 2
NS = 16
NW = NC * NS

ROWS = B * M
RPW = ROWS // NW
CHUNK = 32
NCH = RPW // CHUNK
IPC = CHUNK * S
NSTR = IPC // 128


def _pack_tables(A1, A2, C_last):
  def rne16(xi):
    return lax.shift_right_logical(
        xi + 0x7FFF + (lax.shift_right_logical(xi, 16) & 1), 16)

  cols = []
  for t in (A1, A2, C_last):
    xi = lax.bitcast_convert_type(t, jnp.int32)
    cols.append(rne16(xi[:, :16]) | (rne16(xi[:, 16:]) << 16))
  return jnp.concatenate(cols, axis=1)


def _seg_gather_body(ctx_hbm, tbl_hbm, o_hbm,
                     idx_v, rows_v, out_v, gsem0, gsem1, isem):
  wid = lax.axis_index("s") * NC + lax.axis_index("c")
  idx_base = wid * (NCH * IPC)

  def load_idx(c, buf):
    pltpu.async_copy(ctx_hbm.at[pl.ds(idx_base + c * IPC, IPC)],
                     idx_v.at[buf], isem)

  def wait_idx(buf):
    pltpu.make_async_copy(ctx_hbm.at[pl.ds(0, IPC)], idx_v.at[buf],
                          isem).wait()

  def fire(buf, sem):
    for k in range(NSTR):
      pltpu.async_copy(tbl_hbm.at[idx_v.at[buf, pl.ds(k * 128, 128)]],
                       rows_v.at[buf, pl.ds(k * 128, 128), :],
                       sem)

  def drain(buf, sem):
    for k in range(NSTR):
      pltpu.make_async_copy(tbl_hbm.at[pl.ds(0, 128), :],
                            rows_v.at[buf, pl.ds(k * 128, 128), :],
                            sem).wait()

  hi_mask = jnp.full((16,), -65536, dtype=jnp.int32)

  def widen(xi):
    lo = lax.bitcast_convert_type(xi << 16, jnp.float32)
    hi = lax.bitcast_convert_type(xi & hi_mask, jnp.float32)
    return lo, hi

  def compute_store(c, buf):
    def row(r, carry):
      base = r * S
      for t in range(NT):
        acc0, acc1 = widen(rows_v[buf, base, pl.ds(16 * t, 16)])
        for s in range(1, S):
          lo, hi = widen(rows_v[buf, base + s, pl.ds(16 * t, 16)])
          acc0 = acc0 + lo
          acc1 = acc1 + hi
        out_v[t, r, pl.ds(0, 16)] = acc0
        out_v[t, r, pl.ds(16, 16)] = acc1
      return carry
    lax.fori_loop(0, CHUNK, row, 0)
    pltpu.sync_copy(out_v,
                    o_hbm.at[:, pl.ds(wid * RPW + c * CHUNK, CHUNK), :])

  pltpu.sync_copy(ctx_hbm.at[pl.ds(idx_base, IPC)], idx_v.at[0])
  fire(0, gsem0)
  load_idx(1, 1)

  def pair(j, carry):
    c0 = 2 * j
    wait_idx(1)
    fire(1, gsem1)
    drain(0, gsem0)
    load_idx(c0 + 2, 0)
    compute_store(c0, 0)
    wait_idx(0)
    fire(0, gsem0)
    drain(1, gsem1)
    load_idx(c0 + 3, 1)
    compute_store(c0 + 1, 1)
    return carry

  lax.fori_loop(0, NCH // 2 - 1, pair, 0)
  c0 = NCH - 2
  wait_idx(1)
  fire(1, gsem1)
  drain(0, gsem0)
  compute_store(c0, 0)
  drain(1, gsem1)
  compute_store(c0 + 1, 1)


@functools.cache
def _seg_gather():
  mesh = plsc.VectorSubcoreMesh(
      core_axis_name="c", subcore_axis_name="s",
      num_cores=NC, num_subcores=NS)
  return pl.kernel(
      _seg_gather_body,
      mesh=mesh,
      out_type=jax.ShapeDtypeStruct((NT, ROWS, E), jnp.float32),
      scratch_types=[
          pltpu.VMEM((2, IPC), jnp.int32),
          pltpu.VMEM((2, IPC, PW), jnp.int32),
          pltpu.VMEM((NT, CHUNK, E), jnp.float32),
          pltpu.SemaphoreType.DMA,
          pltpu.SemaphoreType.DMA,
          pltpu.SemaphoreType.DMA,
      ],
      compiler_params=pltpu.CompilerParams(use_tc_tiling_on_sc=False),
  )


def _hops_body(g, o):
  BB = g.shape[1] // M
  g1v = g[0].reshape(BB, M, E)
  q1 = jnp.sum(g1v, axis=1) * (1.0 / M)
  p1 = jnp.sum(g1v * q1[:, None, :], axis=2)
  a1 = jax.nn.softmax(p1, axis=1)
  g2v = g[1].reshape(BB, M, E)
  q2 = q1 + jnp.sum(a1[:, :, None] * g2v, axis=1)
  p2 = jnp.sum(g2v * q2[:, None, :], axis=2)
  a2 = jax.nn.softmax(p2, axis=1)
  o[...] = q2 + jnp.sum(a2[:, :, None] * g[2].reshape(BB, M, E), axis=1)


def _hops(G):
  BB = 128
  return pl.pallas_call(
      _hops_body,
      grid=(B // BB,),
      in_specs=[pl.BlockSpec((NT, BB * M, E), lambda i: (0, i, 0))],
      out_specs=pl.BlockSpec((BB, E), lambda i: (i, 0)),
      out_shape=jax.ShapeDtypeStruct((B, E), jnp.float32),
  )(G)


def kernel(context, A0, A1, A2, C_last):
  del A0
  ctx = context.reshape(-1)
  packed = _pack_tables(A1, A2, C_last)
  G = _seg_gather()(ctx, packed)
  return _hops(G)
